# baseline (speedup 1.0000x reference)
import numpy as np

import jax
import jax.numpy as jnp
from jax.experimental import pallas as pl
from jax.experimental.pallas import tpu as pltpu
from jax.experimental.shard_map import shard_map
from jax.sharding import Mesh, PartitionSpec as P

_HW = 256
_REAL_MAX = 130050
_T0 = float(np.float32(1.0e12))
_T1 = float(np.nextafter(np.float32(_T0), np.float32(np.inf)))
_T2 = float(np.nextafter(np.float32(_T1), np.float32(np.inf)))
_FILL = 2.0e12
_FILL2 = 4.0e12


def _shift(m, di, dj):
    z_row = jnp.zeros((1, _HW), jnp.float32)
    z_col = jnp.zeros((_HW, 1), jnp.float32)
    if di == 1:
        m = jnp.concatenate([m[1:, :], z_row], axis=0)
    elif di == -1:
        m = jnp.concatenate([z_row, m[:-1, :]], axis=0)
    if dj == 1:
        m = jnp.concatenate([m[:, 1:], z_col], axis=1)
    elif dj == -1:
        m = jnp.concatenate([z_col, m[:, :-1]], axis=1)
    return m


def _boundary(m):
    er = m * _shift(m, 1, 0) * _shift(m, -1, 0) * _shift(m, 0, 1) * _shift(m, 0, -1)
    return m * (1.0 - er)


def _edt_pass(a_ref, i2x_ref, dst3_ref):
    def body(t, carry):
        k0 = 2 * t
        k0f = k0.astype(jnp.float32)
        k1f = k0f + 1.0
        i2x = i2x_ref[...]
        v0 = a_ref[...] - k0f * i2x
        r0 = jnp.min(v0, axis=0, keepdims=True) + k0f * k0f
        dst3_ref[pl.ds(k0, 1)] = r0.reshape(1, 1, _HW)
        v1 = v0 - i2x
        r1 = jnp.min(v1, axis=0, keepdims=True) + k1f * k1f
        dst3_ref[pl.ds(k0 + 1, 1)] = r1.reshape(1, 1, _HW)
        return carry
    jax.lax.fori_loop(0, _HW // 2, body, 0)


def _hd95_body(pred_ref, targ_ref, out_ref,
               ct_ref, rn_ref, rt3_ref, d2g3_ref, d2p3_ref,
               epf_ref, egf_ref, i2x_ref, u1_ref, u2_ref):
    x = pred_ref[0, 0]
    pm = jnp.where(jax.nn.sigmoid(x) > 0.5, 1.0, 0.0).astype(jnp.float32)
    gm = jnp.where(targ_ref[0, 0] > 0, 1.0, 0.0).astype(jnp.float32)
    epf_ref[...] = _boundary(pm)
    egf_ref[...] = _boundary(gm)

    isub = jax.lax.broadcasted_iota(jnp.int32, (_HW, _HW), 0).astype(jnp.float32)
    i2x_ref[...] = isub + isub
    isq = isub * isub

    for bnd_ref, d2_ref in ((egf_ref, d2g3_ref), (epf_ref, d2p3_ref)):
        ct_ref[...] = jnp.where(jnp.transpose(bnd_ref[...]) > 0.5,
                                0.0, jnp.float32(_T0)) + isq
        _edt_pass(ct_ref, i2x_ref, rt3_ref)
        rn_ref[...] = jnp.transpose(rt3_ref[:, 0, :]) + isq
        _edt_pass(rn_ref, i2x_ref, d2_ref)

    u1_ref[...] = jnp.where(epf_ref[...] > 0.5, d2g3_ref[:, 0, :],
                            jnp.float32(_FILL))
    u2_ref[...] = jnp.where(egf_ref[...] > 0.5, d2p3_ref[:, 0, :],
                            jnp.float32(_FILL))

    def counts(thr1, thr2):
        c1 = jnp.sum(jnp.where(u1_ref[...] <= thr1, 1.0, 0.0),
                     axis=(0, 1), keepdims=True)
        c2 = jnp.sum(jnp.where(u2_ref[...] <= thr2, 1.0, 0.0),
                     axis=(0, 1), keepdims=True)
        return c1, c2

    def ranks(mask_ref):
        nf = jnp.sum(mask_ref[...], axis=(0, 1), keepdims=True)
        pos = 0.95 * jnp.maximum(nf - 1.0, 0.0)
        lof = jnp.floor(pos)
        frac = pos - lof
        kp1_lo = jnp.clip(lof, 0.0, 65535.0) + 1.0
        kp1_hi = jnp.clip(lof + 1.0, 0.0, 65535.0) + 1.0
        return kp1_lo, kp1_hi, frac

    kl1, kh1, frac1 = ranks(epf_ref)
    kl2, kh2, frac2 = ranks(egf_ref)

    def sbody(_, st):
        lo1, hi1, lo2, hi2 = st
        mid1 = lo1 + jax.lax.shift_right_logical(hi1 - lo1, 1)
        mid2 = lo2 + jax.lax.shift_right_logical(hi2 - lo2, 1)
        c1, c2 = counts(mid1.astype(jnp.float32), mid2.astype(jnp.float32))
        ge1 = c1 >= kl1
        ge2 = c2 >= kl2
        return (jnp.where(ge1, lo1, mid1 + 1), jnp.where(ge1, mid1, hi1),
                jnp.where(ge2, lo2, mid2 + 1), jnp.where(ge2, mid2, hi2))

    z = jnp.zeros((1, 1), jnp.int32)
    hi0 = jnp.full((1, 1), _REAL_MAX, jnp.int32)
    lo1, _, lo2, _ = jax.lax.fori_loop(0, 17, sbody, (z, hi0, z, hi0))

    creal1, creal2 = counts(float(_REAL_MAX), float(_REAL_MAX))
    ct01, ct02 = counts(_T0, _T0)
    ct11, ct12 = counts(_T1, _T1)
    ct21, ct22 = counts(_T2, _T2)

    def pick_sq(lo_int, kp1, creal, ct0, ct1, ct2):
        tier = jnp.where(kp1 <= ct0, _T0,
                         jnp.where(kp1 <= ct1, _T1,
                                   jnp.where(kp1 <= ct2, _T2, _FILL)))
        return jnp.where(kp1 <= creal, lo_int.astype(jnp.float32), tier)

    slo1 = pick_sq(lo1, kl1, creal1, ct01, ct11, ct21)
    slo2 = pick_sq(lo2, kl2, creal2, ct02, ct12, ct22)

    cle1, cle2 = counts(slo1, slo2)
    nx1 = jnp.min(jnp.where(u1_ref[...] > slo1, u1_ref[...], _FILL2),
                  axis=(0, 1), keepdims=True)
    nx2 = jnp.min(jnp.where(u2_ref[...] > slo2, u2_ref[...], _FILL2),
                  axis=(0, 1), keepdims=True)
    shi1 = jnp.where(cle1 >= kh1, slo1, nx1)
    shi2 = jnp.where(cle2 >= kh2, slo2, nx2)

    def dist(sq):
        return jnp.where(sq >= _FILL, jnp.float32(_T0), jnp.sqrt(sq))

    d_pg = dist(slo1) * (1.0 - frac1) + dist(shi1) * frac1
    d_gp = dist(slo2) * (1.0 - frac2) + dist(shi2) * frac2
    out_ref[...] = jnp.maximum(d_pg, d_gp).reshape(1, 1, 1)


_SCRATCHES = [
    pltpu.VMEM((_HW, _HW), jnp.float32),
    pltpu.VMEM((_HW, _HW), jnp.float32),
    pltpu.VMEM((_HW, 1, _HW), jnp.float32),
    pltpu.VMEM((_HW, 1, _HW), jnp.float32),
    pltpu.VMEM((_HW, 1, _HW), jnp.float32),
    pltpu.VMEM((_HW, _HW), jnp.float32),
    pltpu.VMEM((_HW, _HW), jnp.float32),
    pltpu.VMEM((_HW, _HW), jnp.float32),
    pltpu.VMEM((_HW, _HW), jnp.float32),
    pltpu.VMEM((_HW, _HW), jnp.float32),
]


def _build(n_img, interpret=False):
    return pl.pallas_call(
        _hd95_body,
        grid=(n_img,),
        in_specs=[
            pl.BlockSpec((1, 1, _HW, _HW), lambda n: (n, 0, 0, 0)),
            pl.BlockSpec((1, 1, _HW, _HW), lambda n: (n, 0, 0, 0)),
        ],
        out_specs=pl.BlockSpec((1, 1, 1), lambda n: (n, 0, 0)),
        out_shape=jax.ShapeDtypeStruct((n_img, 1, 1), jnp.float32),
        scratch_shapes=_SCRATCHES,
        compiler_params=pltpu.CompilerParams(
            dimension_semantics=("arbitrary",),
        ),
        name="hd95_loss",
        interpret=interpret,
    )


def kernel(pred, target):
    n_img = pred.shape[0] * pred.shape[1]
    n_shards = max(d for d in (1, 2, 4) if d <= jax.device_count() and n_img % d == 0)
    mesh = Mesh(np.asarray(jax.devices()[:n_shards]), ("x",))
    call = _build(n_img // n_shards)
    hd = shard_map(call, mesh=mesh, in_specs=(P("x"), P("x")),
                   out_specs=P("x"), check_rep=False)(pred, target)
    return jnp.mean(hd)


# speedup vs baseline: 3.4700x; 3.4700x over previous
import numpy as np

import jax
import jax.numpy as jnp
from jax.experimental import pallas as pl
from jax.experimental.pallas import tpu as pltpu
_HW = 256
_REAL_MAX = 130050
_T0 = float(np.float32(1.0e12))
_T1 = float(np.nextafter(np.float32(_T0), np.float32(np.inf)))
_T2 = float(np.nextafter(np.float32(_T1), np.float32(np.inf)))
_FILL = 2.0e12
_FILL2 = 4.0e12


def _shift(m, di, dj):
    z_row = jnp.zeros((1, _HW), jnp.float32)
    z_col = jnp.zeros((_HW, 1), jnp.float32)
    if di == 1:
        m = jnp.concatenate([m[1:, :], z_row], axis=0)
    elif di == -1:
        m = jnp.concatenate([z_row, m[:-1, :]], axis=0)
    if dj == 1:
        m = jnp.concatenate([m[:, 1:], z_col], axis=1)
    elif dj == -1:
        m = jnp.concatenate([z_col, m[:, :-1]], axis=1)
    return m


def _boundary(m):
    er = m * _shift(m, 1, 0) * _shift(m, -1, 0) * _shift(m, 0, 1) * _shift(m, 0, -1)
    return m * (1.0 - er)


def _edt_pass_pair(a1_ref, a2_ref, i2x_ref, dst1_ref, dst2_ref):
    def body(t, carry):
        k0 = 2 * t
        k0f = k0.astype(jnp.float32)
        k1f = k0f + 1.0
        i2x = i2x_ref[...]
        step = k0f * i2x
        va0 = a1_ref[...] - step
        vb0 = a2_ref[...] - step
        ra0 = jnp.min(va0, axis=0, keepdims=True) + k0f * k0f
        rb0 = jnp.min(vb0, axis=0, keepdims=True) + k0f * k0f
        dst1_ref[pl.ds(k0, 1)] = ra0.reshape(1, 1, _HW)
        dst2_ref[pl.ds(k0, 1)] = rb0.reshape(1, 1, _HW)
        va1 = va0 - i2x
        vb1 = vb0 - i2x
        ra1 = jnp.min(va1, axis=0, keepdims=True) + k1f * k1f
        rb1 = jnp.min(vb1, axis=0, keepdims=True) + k1f * k1f
        dst1_ref[pl.ds(k0 + 1, 1)] = ra1.reshape(1, 1, _HW)
        dst2_ref[pl.ds(k0 + 1, 1)] = rb1.reshape(1, 1, _HW)
        return carry
    jax.lax.fori_loop(0, _HW // 2, body, 0)


def _hd95_body(pred_ref, targ_ref, out_ref,
               ctg_ref, ctp_ref, rtg3_ref, rtp3_ref, d2g3_ref, d2p3_ref,
               epf_ref, egf_ref, i2x_ref, u1_ref, u2_ref):
    x = pred_ref[0, 0]
    pm = jnp.where(jax.nn.sigmoid(x) > 0.5, 1.0, 0.0).astype(jnp.float32)
    gm = jnp.where(targ_ref[0, 0] > 0, 1.0, 0.0).astype(jnp.float32)
    epf_ref[...] = _boundary(pm)
    egf_ref[...] = _boundary(gm)

    isub = jax.lax.broadcasted_iota(jnp.int32, (_HW, _HW), 0).astype(jnp.float32)
    i2x_ref[...] = isub + isub
    isq = isub * isub

    ctg_ref[...] = jnp.where(jnp.transpose(egf_ref[...]) > 0.5,
                             0.0, jnp.float32(_T0)) + isq
    ctp_ref[...] = jnp.where(jnp.transpose(epf_ref[...]) > 0.5,
                             0.0, jnp.float32(_T0)) + isq
    _edt_pass_pair(ctg_ref, ctp_ref, i2x_ref, rtg3_ref, rtp3_ref)
    ctg_ref[...] = jnp.transpose(rtg3_ref[:, 0, :]) + isq
    ctp_ref[...] = jnp.transpose(rtp3_ref[:, 0, :]) + isq
    _edt_pass_pair(ctg_ref, ctp_ref, i2x_ref, d2g3_ref, d2p3_ref)

    u1_ref[...] = jnp.where(epf_ref[...] > 0.5, d2g3_ref[:, 0, :],
                            jnp.float32(_FILL))
    u2_ref[...] = jnp.where(egf_ref[...] > 0.5, d2p3_ref[:, 0, :],
                            jnp.float32(_FILL))

    def counts(thr1, thr2):
        c1 = jnp.sum(jnp.where(u1_ref[...] <= thr1, 1.0, 0.0),
                     axis=(0, 1), keepdims=True)
        c2 = jnp.sum(jnp.where(u2_ref[...] <= thr2, 1.0, 0.0),
                     axis=(0, 1), keepdims=True)
        return c1, c2

    def ranks(mask_ref):
        nf = jnp.sum(mask_ref[...], axis=(0, 1), keepdims=True)
        pos = 0.95 * jnp.maximum(nf - 1.0, 0.0)
        lof = jnp.floor(pos)
        frac = pos - lof
        kp1_lo = jnp.clip(lof, 0.0, 65535.0) + 1.0
        kp1_hi = jnp.clip(lof + 1.0, 0.0, 65535.0) + 1.0
        return kp1_lo, kp1_hi, frac

    kl1, kh1, frac1 = ranks(epf_ref)
    kl2, kh2, frac2 = ranks(egf_ref)

    def sbody(_, st):
        lo1, hi1, lo2, hi2 = st
        mid1 = lo1 + jax.lax.shift_right_logical(hi1 - lo1, 1)
        mid2 = lo2 + jax.lax.shift_right_logical(hi2 - lo2, 1)
        c1, c2 = counts(mid1.astype(jnp.float32), mid2.astype(jnp.float32))
        ge1 = c1 >= kl1
        ge2 = c2 >= kl2
        return (jnp.where(ge1, lo1, mid1 + 1), jnp.where(ge1, mid1, hi1),
                jnp.where(ge2, lo2, mid2 + 1), jnp.where(ge2, mid2, hi2))

    z = jnp.zeros((1, 1), jnp.int32)
    hi0 = jnp.full((1, 1), _REAL_MAX, jnp.int32)
    lo1, _, lo2, _ = jax.lax.fori_loop(0, 17, sbody, (z, hi0, z, hi0))

    creal1, creal2 = counts(float(_REAL_MAX), float(_REAL_MAX))
    ct01, ct02 = counts(_T0, _T0)
    ct11, ct12 = counts(_T1, _T1)
    ct21, ct22 = counts(_T2, _T2)

    def pick_sq(lo_int, kp1, creal, ct0, ct1, ct2):
        tier = jnp.where(kp1 <= ct0, _T0,
                         jnp.where(kp1 <= ct1, _T1,
                                   jnp.where(kp1 <= ct2, _T2, _FILL)))
        return jnp.where(kp1 <= creal, lo_int.astype(jnp.float32), tier)

    slo1 = pick_sq(lo1, kl1, creal1, ct01, ct11, ct21)
    slo2 = pick_sq(lo2, kl2, creal2, ct02, ct12, ct22)

    cle1, cle2 = counts(slo1, slo2)
    nx1 = jnp.min(jnp.where(u1_ref[...] > slo1, u1_ref[...], _FILL2),
                  axis=(0, 1), keepdims=True)
    nx2 = jnp.min(jnp.where(u2_ref[...] > slo2, u2_ref[...], _FILL2),
                  axis=(0, 1), keepdims=True)
    shi1 = jnp.where(cle1 >= kh1, slo1, nx1)
    shi2 = jnp.where(cle2 >= kh2, slo2, nx2)

    def dist(sq):
        return jnp.where(sq >= _FILL, jnp.float32(_T0), jnp.sqrt(sq))

    d_pg = dist(slo1) * (1.0 - frac1) + dist(shi1) * frac1
    d_gp = dist(slo2) * (1.0 - frac2) + dist(shi2) * frac2
    out_ref[...] = jnp.maximum(d_pg, d_gp).reshape(1, 1, 1)


_SCRATCHES = [
    pltpu.VMEM((_HW, _HW), jnp.float32),
    pltpu.VMEM((_HW, _HW), jnp.float32),
    pltpu.VMEM((_HW, 1, _HW), jnp.float32),
    pltpu.VMEM((_HW, 1, _HW), jnp.float32),
    pltpu.VMEM((_HW, 1, _HW), jnp.float32),
    pltpu.VMEM((_HW, 1, _HW), jnp.float32),
    pltpu.VMEM((_HW, _HW), jnp.float32),
    pltpu.VMEM((_HW, _HW), jnp.float32),
    pltpu.VMEM((_HW, _HW), jnp.float32),
    pltpu.VMEM((_HW, _HW), jnp.float32),
    pltpu.VMEM((_HW, _HW), jnp.float32),
]


def _build(n_img, interpret=False):
    return pl.pallas_call(
        _hd95_body,
        grid=(n_img,),
        in_specs=[
            pl.BlockSpec((1, 1, _HW, _HW), lambda n: (n, 0, 0, 0)),
            pl.BlockSpec((1, 1, _HW, _HW), lambda n: (n, 0, 0, 0)),
        ],
        out_specs=pl.BlockSpec((1, 1, 1), lambda n: (n, 0, 0)),
        out_shape=jax.ShapeDtypeStruct((n_img, 1, 1), jnp.float32),
        scratch_shapes=_SCRATCHES,
        compiler_params=pltpu.CompilerParams(
            dimension_semantics=("arbitrary",),
        ),
        name="hd95_loss",
        interpret=interpret,
    )


def kernel(pred, target):
    hd = _build(pred.shape[0] * pred.shape[1])(pred, target)
    return jnp.mean(hd)
